# Initial kernel scaffold; baseline (speedup 1.0000x reference)
import jax
import jax.numpy as jnp
from jax.experimental import pallas as pl


def kernel(feature, xyz, neigh_idx, W1, g1, b1, Wb1, gb1, bb1, Wf1, Wm1, gm1, bm1, Wb2, gb2, bb2, Wf2, Wm2, gm2, bm2, W2, g2, b2, Ws, gs, bs):
    raise NotImplementedError("write your pallas kernel here")


# speedup vs baseline: 11.1892x; 11.1892x over previous
import functools

import jax
import jax.numpy as jnp
from jax import lax
from jax.experimental import pallas as pl
from jax.experimental.pallas import tpu as pltpu
from jax.experimental.pallas import tpu_sc as plsc

NPAD = 10240
TN = 512
KNN = 16
NC, NS = 2, 16
NW = NC * NS
CH = 128


def _mm(x, w):
    return lax.dot_general(x, w, (((1,), (1,)), ((), ())),
                           preferred_element_type=jnp.float32)


def _make_gather(rows_total, d):
    rows_per_w = rows_total // NW
    n_ch = rows_per_w // CH
    mesh = plsc.VectorSubcoreMesh(core_axis_name="c", subcore_axis_name="s")

    @functools.partial(
        pl.kernel,
        out_type=jax.ShapeDtypeStruct((rows_total, d), jnp.float32),
        mesh=mesh,
        scratch_types=[
            pltpu.VMEM((rows_per_w,), jnp.int32),
            pltpu.VMEM((CH, d), jnp.float32),
            pltpu.SemaphoreType.DMA,
        ],
    )
    def gather(tbl_hbm, idx_hbm, out_hbm, idx_v, buf, sem):
        wid = lax.axis_index("s") * NC + lax.axis_index("c")
        base = wid * rows_per_w
        pltpu.sync_copy(idx_hbm.at[pl.ds(base, rows_per_w)], idx_v)

        def body(i, carry):
            off = pl.multiple_of(i * CH, CH)
            pltpu.async_copy(tbl_hbm.at[idx_v.at[pl.ds(off, CH)]], buf,
                             sem).wait()
            pltpu.sync_copy(buf, out_hbm.at[pl.ds(base + off, CH)])
            return carry

        lax.fori_loop(0, n_ch, body, 0)

    return gather


def _kernel_a(feat_ref, xyz_ref, w1_ref, b1_ref, out_ref):
    x = feat_ref[0]
    pc1 = jnp.maximum(_mm(x, w1_ref[...]) + b1_ref[...], 0.0)
    out_ref[0] = jnp.concatenate(
        [pc1, xyz_ref[0], jnp.zeros((TN, 48), jnp.float32)], axis=-1)


def _fxyz1(nb, xyz_t, wa_ref, wb_ref, wdis_ref, bb1_ref):
    nxyz = nb[:, :, 64:80]
    rel = xyz_t[None] - nxyz
    ss = jnp.sum(rel * rel, axis=-1, keepdims=True)
    dis = jnp.sqrt(jnp.maximum(ss, 1e-20))
    pbase = _mm(xyz_t, wa_ref[...]) + bb1_ref[...]
    e = _mm(nxyz.reshape(KNN * TN, 16), wb_ref[...]).reshape(KNN, TN, 64)
    return jnp.maximum(dis * wdis_ref[...][None] + e + pbase[None], 0.0)


def _att_pool(x, att):
    m = jnp.max(att, axis=0)
    e = jnp.exp(att - m[None])
    return jnp.sum(x * e, axis=0) / jnp.sum(e, axis=0)


def _kernel_b(nb_ref, xyz_ref, wa_ref, wb_ref, wdis_ref, bb1_ref,
              wf1_ref, wm1_ref, bm1_ref, out_ref):
    nb = nb_ref[0]
    f1 = _fxyz1(nb, xyz_ref[0], wa_ref, wb_ref, wdis_ref, bb1_ref)
    x = jnp.concatenate([nb[:, :, :64], f1], axis=-1)
    att = _mm(x.reshape(KNN * TN, 128), wf1_ref[...]).reshape(KNN, TN, 128)
    f = _att_pool(x, att)
    fa1 = jnp.maximum(_mm(f, wm1_ref[...]) + bm1_ref[...], 0.0)
    out_ref[0] = jnp.concatenate([fa1, jnp.zeros((TN, 64), jnp.float32)],
                                 axis=-1)


def _kernel_c(nb_ref, fn2_ref, xyz_ref, feat_ref, wa_ref, wb_ref, wdis_ref,
              bb1_ref, wb2_ref, bb2_ref, wf2_ref, wm2_ref, bm2_ref,
              w2_ref, b2_ref, ws_ref, bs_ref, out_ref):
    nb = nb_ref[0]
    f1 = _fxyz1(nb, xyz_ref[0], wa_ref, wb_ref, wdis_ref, bb1_ref)
    f2 = jnp.maximum(
        _mm(f1.reshape(KNN * TN, 64), wb2_ref[...]) + bb2_ref[...], 0.0)
    x2 = jnp.concatenate([fn2_ref[0][:, :, :64], f2.reshape(KNN, TN, 64)],
                         axis=-1)
    att = _mm(x2.reshape(KNN * TN, 128), wf2_ref[...]).reshape(KNN, TN, 128)
    fp = _att_pool(x2, att)
    fa2 = jnp.maximum(_mm(fp, wm2_ref[...]) + bm2_ref[...], 0.0)
    y = (_mm(fa2, w2_ref[...]) + b2_ref[...]
         + _mm(feat_ref[0], ws_ref[...]) + bs_ref[...])
    out_ref[0] = jnp.where(y > 0, y, 0.2 * y)


def _full(shape):
    return pl.BlockSpec(shape, lambda b, t: (0,) * len(shape))


@jax.jit
def kernel(feature, xyz, neigh_idx, W1, g1, b1, Wb1, gb1, bb1, Wf1, Wm1, gm1,
           bm1, Wb2, gb2, bb2, Wf2, Wm2, gm2, bm2, W2, g2, b2, Ws, gs, bs):
    B = feature.shape[0]
    N = feature.shape[2]
    NT = NPAD // TN
    R = B * KNN * NPAD

    featT = jnp.pad(jnp.squeeze(feature, -1).transpose(0, 2, 1),
                    ((0, 0), (0, NPAD - N), (0, 0)))
    xyzp = jnp.pad(xyz, ((0, 0), (0, NPAD - N), (0, 13)))
    idx = neigh_idx.astype(jnp.int32).transpose(0, 2, 1)
    idx = jnp.pad(idx, ((0, 0), (0, 0), (0, NPAD - N)))
    idx = idx + (jnp.arange(B, dtype=jnp.int32) * NPAD)[:, None, None]
    idx = idx.reshape(R)

    w1f = W1 * g1[:, None]
    wb1f = Wb1 * gb1[:, None]
    pad3 = lambda w: jnp.pad(w, ((0, 0), (0, 13)))
    wa = pad3(wb1f[:, 1:4] + wb1f[:, 4:7])
    wb = pad3(wb1f[:, 7:10] - wb1f[:, 1:4])
    wdis = wb1f[:, 0][None, :]
    wb2f = Wb2 * gb2[:, None]
    wm1f = Wm1 * gm1[:, None]
    wm2f = Wm2 * gm2[:, None]
    w2f = W2 * g2[:, None]
    wsf = Ws * gs[:, None]
    row = lambda v: v[None, :].astype(jnp.float32)

    tbl1 = pl.pallas_call(
        _kernel_a,
        grid=(B, NT),
        in_specs=[
            pl.BlockSpec((1, TN, 128), lambda b, t: (b, t, 0)),
            pl.BlockSpec((1, TN, 16), lambda b, t: (b, t, 0)),
            _full((64, 128)), _full((1, 64)),
        ],
        out_specs=pl.BlockSpec((1, TN, 128), lambda b, t: (b, t, 0)),
        out_shape=jax.ShapeDtypeStruct((B, NPAD, 128), jnp.float32),
    )(featT, xyzp, w1f, row(b1))

    g1out = _make_gather(R, 128)(tbl1.reshape(B * NPAD, 128), idx)
    nb = g1out.reshape(B, KNN, NPAD, 128)

    fa1 = pl.pallas_call(
        _kernel_b,
        grid=(B, NT),
        in_specs=[
            pl.BlockSpec((1, KNN, TN, 128), lambda b, t: (b, 0, t, 0)),
            pl.BlockSpec((1, TN, 16), lambda b, t: (b, t, 0)),
            _full((64, 16)), _full((64, 16)), _full((1, 64)), _full((1, 64)),
            _full((128, 128)), _full((64, 128)), _full((1, 64)),
        ],
        out_specs=pl.BlockSpec((1, TN, 128), lambda b, t: (b, t, 0)),
        out_shape=jax.ShapeDtypeStruct((B, NPAD, 128), jnp.float32),
    )(nb, xyzp, wa, wb, wdis, row(bb1), Wf1, wm1f, row(bm1))

    g2out = _make_gather(R, 128)(fa1.reshape(B * NPAD, 128), idx)
    fn2 = g2out.reshape(B, KNN, NPAD, 128)

    out = pl.pallas_call(
        _kernel_c,
        grid=(B, NT),
        in_specs=[
            pl.BlockSpec((1, KNN, TN, 128), lambda b, t: (b, 0, t, 0)),
            pl.BlockSpec((1, KNN, TN, 128), lambda b, t: (b, 0, t, 0)),
            pl.BlockSpec((1, TN, 16), lambda b, t: (b, t, 0)),
            pl.BlockSpec((1, TN, 128), lambda b, t: (b, t, 0)),
            _full((64, 16)), _full((64, 16)), _full((1, 64)), _full((1, 64)),
            _full((64, 64)), _full((1, 64)),
            _full((128, 128)), _full((128, 128)), _full((1, 128)),
            _full((256, 128)), _full((1, 256)), _full((256, 128)),
            _full((1, 256)),
        ],
        out_specs=pl.BlockSpec((1, TN, 256), lambda b, t: (b, t, 0)),
        out_shape=jax.ShapeDtypeStruct((B, NPAD, 256), jnp.float32),
    )(nb, fn2, xyzp, featT, wa, wb, wdis, row(bb1), wb2f, row(bb2),
      Wf2, wm2f, row(bm2), w2f, row(b2), wsf, row(bs))

    return out[:, :N, :].transpose(0, 2, 1)[..., None]
